# baseline (speedup 1.0000x reference)
import functools

import jax
import jax.numpy as jnp
from jax import lax
from jax.experimental import pallas as pl
from jax.experimental.pallas import tpu as pltpu
from jax.experimental.pallas import tpu_sc as plsc

BATCH = 16384
HIST = 50
D = 32
NC = 2
NS = 16
NW = NC * NS
IPR = 128
BT_PER_W = (BATCH // IPR) // NW
UNITS = HIST * BT_PER_W
DHI = D // 8
NBUF = 4


def _embed_call(idx_t, table):
    mesh = plsc.VectorSubcoreMesh(core_axis_name="c", subcore_axis_name="s")

    @functools.partial(
        pl.kernel,
        mesh=mesh,
        out_type=jax.ShapeDtypeStruct((HIST, DHI, BATCH // IPR, 8, IPR),
                                      jnp.float32),
        compiler_params=pltpu.CompilerParams(
            use_tc_tiling_on_sc=False, needs_layout_passes=False
        ),
        scratch_types=[
            pltpu.VMEM((HIST, BT_PER_W * IPR), jnp.int32),
            pltpu.VMEM((NBUF, IPR, D), jnp.float32),
            pltpu.VMEM((NBUF, D, IPR), jnp.float32),
            [pltpu.SemaphoreType.DMA] * NBUF,
            [pltpu.SemaphoreType.DMA] * NBUF,
        ],
    )
    def k(idx_hbm, table_hbm, out_hbm, idx_v, rows_v, tsp_v, gsems, ssems):
        wid = lax.axis_index("s") * NC + lax.axis_index("c")
        bt_base = wid * BT_PER_W
        pltpu.sync_copy(idx_hbm.at[:, pl.ds(bt_base * IPR, BT_PER_W * IPR)],
                        idx_v)

        lane = lax.iota(jnp.int32, 16)
        row_bases = [lane + 16 * kk for kk in range(8)]

        def issue_gather(u, b):
            h = u // BT_PER_W
            t = u % BT_PER_W
            pltpu.async_copy(
                table_hbm.at[idx_v.at[h, pl.ds(t * IPR, IPR)]],
                rows_v.at[b],
                gsems[b],
            )

        def drain_gather(b):
            pltpu.make_async_copy(
                table_hbm.at[pl.ds(0, IPR)], rows_v.at[b], gsems[b]
            ).wait()

        def transpose(b):
            @plsc.parallel_loop(0, D, 1, unroll=4)
            def _(d):
                col = jnp.full((16,), 0, jnp.int32) + d
                for kk in range(8):
                    v = plsc.load_gather(rows_v.at[b], [row_bases[kk], col])
                    tsp_v[b, d, pl.ds(16 * kk, 16)] = v

        def issue_stores(u, b):
            h = u // BT_PER_W
            t = u % BT_PER_W
            for kk in range(DHI):
                pltpu.async_copy(
                    tsp_v.at[b, pl.ds(8 * kk, 8)],
                    out_hbm.at[h, kk, bt_base + t],
                    ssems[b],
                )

        def drain_stores(b):
            for kk in range(DHI):
                pltpu.make_async_copy(
                    tsp_v.at[b, pl.ds(8 * kk, 8)], out_hbm.at[0, kk, 0],
                    ssems[b],
                ).wait()

        for b in range(NBUF):
            issue_gather(b, b)

        def ring(s, carry):
            for b in range(NBUF):
                u = s * NBUF + b
                drain_gather(b)

                @pl.when(s >= 1)
                def _():
                    drain_stores(b)

                transpose(b)
                issue_stores(u, b)

                @pl.when(s < UNITS // NBUF - 1)
                def _():
                    issue_gather(u + NBUF, b)

            return carry

        lax.fori_loop(0, UNITS // NBUF, ring, 0)
        for b in range(NBUF):
            drain_stores(b)

    return k(idx_t, table)


def kernel(inputs, table):
    idx_t = inputs.T
    o6 = _embed_call(idx_t, table)
    return o6.transpose(2, 4, 0, 1, 3).reshape(BATCH, HIST, D)


# speedup vs baseline: 1.0070x; 1.0070x over previous
import functools

import jax
import jax.numpy as jnp
from jax import lax
from jax.experimental import pallas as pl
from jax.experimental.pallas import tpu as pltpu
from jax.experimental.pallas import tpu_sc as plsc

BATCH = 16384
HIST = 50
D = 32
NC = 2
NS = 16
NW = NC * NS
IPR = 128
BT_PER_W = (BATCH // IPR) // NW
GROUP = BT_PER_W * IPR
DHI = D // 8
NBUF = 2


def _embed_call(idx_t, table):
    mesh = plsc.VectorSubcoreMesh(core_axis_name="c", subcore_axis_name="s")

    @functools.partial(
        pl.kernel,
        mesh=mesh,
        out_type=jax.ShapeDtypeStruct((HIST, DHI, BATCH // IPR, 8, IPR),
                                      jnp.float32),
        compiler_params=pltpu.CompilerParams(
            use_tc_tiling_on_sc=False, needs_layout_passes=False
        ),
        scratch_types=[
            pltpu.VMEM((HIST, GROUP), jnp.int32),
            pltpu.VMEM((NBUF, GROUP, D), jnp.float32),
            pltpu.VMEM((NBUF, DHI, BT_PER_W, 8, IPR), jnp.float32),
            [pltpu.SemaphoreType.DMA] * NBUF,
            [pltpu.SemaphoreType.DMA] * NBUF,
        ],
    )
    def k(idx_hbm, table_hbm, out_hbm, idx_v, rows_v, tsp_v, gsems, ssems):
        wid = lax.axis_index("s") * NC + lax.axis_index("c")
        bt_base = wid * BT_PER_W
        pltpu.sync_copy(idx_hbm.at[:, pl.ds(bt_base * IPR, GROUP)], idx_v)

        lane = lax.iota(jnp.int32, 16)
        row_bases = [lane + 16 * kk for kk in range(8)]

        def issue_gathers(g, b):
            for t in range(BT_PER_W):
                pltpu.async_copy(
                    table_hbm.at[idx_v.at[g, pl.ds(t * IPR, IPR)]],
                    rows_v.at[b, pl.ds(t * IPR, IPR)],
                    gsems[b],
                )

        def drain_gathers(b):
            pltpu.make_async_copy(
                table_hbm.at[pl.ds(0, GROUP)], rows_v.at[b], gsems[b]
            ).wait()

        def transpose(b):
            @plsc.parallel_loop(0, D, 1, unroll=4)
            def _(d):
                col = jnp.full((16,), 0, jnp.int32) + d
                dhi = d // 8
                dlo = d % 8
                for t in range(BT_PER_W):
                    for kk in range(8):
                        v = plsc.load_gather(
                            rows_v.at[b],
                            [row_bases[kk] + t * IPR, col],
                        )
                        tsp_v[b, dhi, t, dlo, pl.ds(16 * kk, 16)] = v

        def issue_stores(g, b):
            for kk in range(DHI):
                pltpu.async_copy(
                    tsp_v.at[b, kk],
                    out_hbm.at[g, kk, pl.ds(bt_base, BT_PER_W)],
                    ssems[b],
                )

        def drain_stores(b):
            for kk in range(DHI):
                pltpu.make_async_copy(
                    tsp_v.at[b, kk],
                    out_hbm.at[0, kk, pl.ds(bt_base, BT_PER_W)],
                    ssems[b],
                ).wait()

        for b in range(NBUF):
            issue_gathers(b, b)

        def ring(s, carry):
            for b in range(NBUF):
                g = s * NBUF + b
                drain_gathers(b)

                @pl.when(s >= 1)
                def _():
                    drain_stores(b)

                transpose(b)
                issue_stores(g, b)

                @pl.when(s < HIST // NBUF - 1)
                def _():
                    issue_gathers(g + NBUF, b)

            return carry

        lax.fori_loop(0, HIST // NBUF, ring, 0)
        for b in range(NBUF):
            drain_stores(b)

    return k(idx_t, table)


def kernel(inputs, table):
    idx_t = inputs.T
    o6 = _embed_call(idx_t, table)
    return o6.transpose(2, 4, 0, 1, 3).reshape(BATCH, HIST, D)
